# baseline (speedup 1.0000x reference)
import functools

import jax
import jax.numpy as jnp
from jax import lax
from jax.experimental import pallas as pl
from jax.experimental.pallas import tpu as pltpu
from jax.experimental.pallas import tpu_sc as plsc

N = 10000
E = 320000
D = 128
H = 128
NLAB = 40
EPS = 1e-05

NC = 2
NS = 16
NW = NC * NS
EW = E // NW
CH = 80
NCHUNK = EW // CH
KBUF = 4
EBUF = 8
NB8 = (NCHUNK - 5) // EBUF
ZR = 80
NRC = N // ZR
GPT = 1024 // NS

_mesh = plsc.VectorSubcoreMesh(core_axis_name="c", subcore_axis_name="s")


def _spmm_body(write_full, x_hbm, row_hbm, col_hbm, val_hbm, idx_hbm, out_hbm,
               acc, ecol, erow, evalv, rows, idxv, *sems):
    esem = sems[:EBUF]
    gsem = sems[EBUF:EBUF + KBUF]
    scsem = sems[EBUF + KBUF:EBUF + 2 * KBUF]
    zsem = sems[EBUF + 2 * KBUF]
    c_ax = lax.axis_index("c")
    s = lax.axis_index("s")
    wid = s * NC + c_ax

    def edata_start(cc, eb):
        pltpu.async_copy(col_hbm.at[wid, cc], ecol.at[eb], esem[eb])
        pltpu.async_copy(row_hbm.at[wid, cc], erow.at[eb], esem[eb])
        pltpu.async_copy(val_hbm.at[wid, cc], evalv.at[eb], esem[eb])

    def edata_wait(eb):
        pltpu.make_async_copy(col_hbm.at[0, 0], ecol.at[eb], esem[eb]).wait()
        pltpu.make_async_copy(row_hbm.at[0, 0], erow.at[eb], esem[eb]).wait()
        pltpu.make_async_copy(val_hbm.at[0, 0], evalv.at[eb], esem[eb]).wait()

    def gather_start(b, eb):
        pltpu.async_copy(x_hbm.at[ecol.at[eb, 0]], rows.at[b], gsem[b])

    def gather_wait(b):
        pltpu.make_async_copy(x_hbm.at[ecol.at[0, 0]], rows.at[b], gsem[b]).wait()

    def scatter_start(b, eb):
        pltpu.async_copy(rows.at[b], acc.at[erow.at[eb, 0]], scsem[b], add=True)

    def scatter_wait(b):
        pltpu.make_async_copy(rows.at[b], acc.at[erow.at[0, 0]], scsem[b]).wait()

    def scale(b, eb):
        rb = rows.at[b]

        @plsc.parallel_loop(0, CH // 16)
        def grp(gg):
            vals16 = evalv[eb, 0, pl.ds(16 * gg, 16)]
            for le in range(16):
                vv = jnp.full((16,), vals16[le], jnp.float32)
                e = 16 * gg + le
                for j in range(H // 16):
                    sl = pl.ds(16 * j, 16)
                    rb[e, sl] = rb[e, sl] * vv

    for eb in range(KBUF):
        edata_start(eb, eb)

    zbuf = rows.at[0]

    def zrow(i, carry):
        for j in range(H // 16):
            zbuf[i, pl.ds(16 * j, 16)] = jnp.zeros((16,), jnp.float32)
        return carry
    lax.fori_loop(0, ZR, zrow, 0)

    def zchunk(k, carry):
        cidx = s + NS * k
        @pl.when(cidx < NRC)
        def _():
            pltpu.async_copy(zbuf, acc.at[pl.ds(ZR * cidx, ZR)], zsem)
        return carry
    lax.fori_loop(0, (NRC + NS - 1) // NS, zchunk, 0)

    def zdrain(k, carry):
        cidx = s + NS * k
        @pl.when(cidx < NRC)
        def _():
            pltpu.make_async_copy(zbuf, acc.at[pl.ds(0, ZR)], zsem).wait()
        return carry
    lax.fori_loop(0, (NRC + NS - 1) // NS, zdrain, 0)
    plsc.subcore_barrier()

    edata_wait(0)
    gather_start(0, 0)
    edata_wait(1)
    gather_start(1, 1)

    def grpblk(g, carry):
        for bb in range(EBUF):
            b = bb % KBUF
            if bb < 2:
                @pl.when(g >= 1)
                def _():
                    scatter_wait((b + 2) % KBUF)
            else:
                scatter_wait((b + 2) % KBUF)
            edata_wait((bb + 2) % EBUF)
            gather_start((b + 2) % KBUF, (bb + 2) % EBUF)
            edata_start(EBUF * g + bb + 4, (bb + 4) % EBUF)
            gather_wait(b)
            scale(b, bb)
            scatter_start(b, bb)
        return carry
    lax.fori_loop(0, NB8, grpblk, 0)

    for cc in range(EBUF * NB8, NCHUNK):
        b = cc % KBUF
        eb = cc % EBUF
        scatter_wait((b + 2) % KBUF)
        if cc + 2 < NCHUNK:
            edata_wait((eb + 2) % EBUF)
            gather_start((b + 2) % KBUF, (eb + 2) % EBUF)
        if cc + 4 < NCHUNK:
            edata_start(cc + 4, (eb + 4) % EBUF)
        gather_wait(b)
        scale(b, eb)
        scatter_start(b, eb)
    scatter_wait(3)
    scatter_wait(0)
    plsc.subcore_barrier()

    if write_full:
        def wchunk(k, carry):
            cidx = s + NS * k
            @pl.when(cidx < NRC)
            def _():
                pltpu.async_copy(acc.at[pl.ds(ZR * cidx, ZR)],
                                 out_hbm.at[c_ax, pl.ds(ZR * cidx, ZR)], zsem)
            return carry
        lax.fori_loop(0, (NRC + NS - 1) // NS, wchunk, 0)

        def wdrain(k, carry):
            cidx = s + NS * k
            @pl.when(cidx < NRC)
            def _():
                pltpu.make_async_copy(acc.at[pl.ds(0, ZR)],
                                      out_hbm.at[c_ax, pl.ds(0, ZR)],
                                      zsem).wait()
            return carry
        lax.fori_loop(0, (NRC + NS - 1) // NS, wdrain, 0)
    else:
        gview = rows.at[0, pl.ds(0, GPT)]
        pltpu.sync_copy(idx_hbm.at[pl.ds(s * GPT, GPT)], idxv)
        pltpu.sync_copy(acc.at[idxv], gview)
        pltpu.sync_copy(gview, out_hbm.at[c_ax, pl.ds(s * GPT, GPT)])


def _make_spmm(write_full):
    out_rows = N if write_full else 1024
    return functools.partial(
        pl.kernel,
        mesh=_mesh,
        out_type=jax.ShapeDtypeStruct((NC, out_rows, H), jnp.float32),
        scratch_types=[
            pltpu.VMEM_SHARED((N, H), jnp.float32),
            pltpu.VMEM((EBUF, 1, CH), jnp.int32),
            pltpu.VMEM((EBUF, 1, CH), jnp.int32),
            pltpu.VMEM((EBUF, 1, CH), jnp.float32),
            pltpu.VMEM((KBUF, CH, H), jnp.float32),
            pltpu.VMEM((GPT,), jnp.int32),
        ] + [pltpu.SemaphoreType.DMA] * (EBUF + 2 * KBUF + 1),
    )(functools.partial(_spmm_body, write_full))


_spmm_full = _make_spmm(True)
_spmm_gather = _make_spmm(False)


def _tc1_body(f_ref, w_ref, b_ref, o_ref):
    o_ref[...] = jnp.dot(f_ref[...], w_ref[...],
                         preferred_element_type=jnp.float32) + b_ref[...]


def _tc2_body(p_ref, s_ref, h_ref, w_ref, o_ref):
    x = p_ref[0] + p_ref[1]
    y = jnp.maximum(x * s_ref[...] + h_ref[...], 0.0)
    o_ref[...] = jnp.dot(y, w_ref[...], preferred_element_type=jnp.float32)


def _tc3_body(g_ref, s_ref, h_ref, w_ref, b_ref, o_ref):
    x = g_ref[0] + g_ref[1]
    y = jnp.maximum(x * s_ref[...] + h_ref[...], 0.0)
    o_ref[...] = jnp.dot(y, w_ref[...],
                         preferred_element_type=jnp.float32) + b_ref[...]


_RB = 2000


def kernel(features, edge_index, edge_vals, idx, W0, b0, bl0, gamma0, beta0,
           mean0, var0, W1, bl1, gamma1, beta1, mean1, var1, Wf, bf):
    row = edge_index[0].reshape(NW, NCHUNK, 1, CH)
    col = edge_index[1].reshape(NW, NCHUNK, 1, CH)
    val = edge_vals.reshape(NW, NCHUNK, 1, CH)

    scale0 = (gamma0 * lax.rsqrt(var0 + EPS)).reshape(1, H)
    shift0 = ((bl0 - mean0) * gamma0 * lax.rsqrt(var0 + EPS) + beta0).reshape(1, H)
    scale1 = (gamma1 * lax.rsqrt(var1 + EPS)).reshape(1, H)
    shift1 = ((bl1 - mean1) * gamma1 * lax.rsqrt(var1 + EPS) + beta1).reshape(1, H)

    x1 = pl.pallas_call(
        _tc1_body,
        grid=(N // _RB,),
        in_specs=[
            pl.BlockSpec((_RB, D), lambda i: (i, 0)),
            pl.BlockSpec((D, H), lambda i: (0, 0)),
            pl.BlockSpec((1, H), lambda i: (0, 0)),
        ],
        out_specs=pl.BlockSpec((_RB, H), lambda i: (i, 0)),
        out_shape=jax.ShapeDtypeStruct((N, H), jnp.float32),
    )(features, W0, b0.reshape(1, H))

    p1 = _spmm_full(x1, row, col, val, idx)

    x2 = pl.pallas_call(
        _tc2_body,
        grid=(N // _RB,),
        in_specs=[
            pl.BlockSpec((NC, _RB, H), lambda i: (0, i, 0)),
            pl.BlockSpec((1, H), lambda i: (0, 0)),
            pl.BlockSpec((1, H), lambda i: (0, 0)),
            pl.BlockSpec((H, H), lambda i: (0, 0)),
        ],
        out_specs=pl.BlockSpec((_RB, H), lambda i: (i, 0)),
        out_shape=jax.ShapeDtypeStruct((N, H), jnp.float32),
    )(p1, scale0, shift0, W1)

    g = _spmm_gather(x2, row, col, val, idx)

    out = pl.pallas_call(
        _tc3_body,
        in_specs=[
            pl.BlockSpec((NC, 1024, H), lambda: (0, 0, 0)),
            pl.BlockSpec((1, H), lambda: (0, 0)),
            pl.BlockSpec((1, H), lambda: (0, 0)),
            pl.BlockSpec((H, NLAB), lambda: (0, 0)),
            pl.BlockSpec((1, NLAB), lambda: (0, 0)),
        ],
        out_specs=pl.BlockSpec((1024, NLAB), lambda: (0, 0)),
        out_shape=jax.ShapeDtypeStruct((1024, NLAB), jnp.float32),
    )(g, scale1, shift1, Wf, bf.reshape(1, NLAB))

    return out


# speedup vs baseline: 1.2316x; 1.2316x over previous
import functools

import jax
import jax.numpy as jnp
from jax import lax
from jax.experimental import pallas as pl
from jax.experimental.pallas import tpu as pltpu
from jax.experimental.pallas import tpu_sc as plsc

N = 10000
E = 320000
D = 128
H = 128
NLAB = 40
EPS = 1e-05

NC = 2
NS = 16
NW = NC * NS
EW = E // NW
CH = 80
NCHUNK = EW // CH
KBUF = 4
EBUF = 8
NB8 = (NCHUNK - 5) // EBUF
ZR = 80
NRC = N // ZR
GPT = 1024 // NS

_mesh = plsc.VectorSubcoreMesh(core_axis_name="c", subcore_axis_name="s")


def _spmm_body(write_full, x_hbm, row_hbm, col_hbm, val_hbm, idx_hbm, out_hbm,
               acc, ecol, erow, evalv, rows, idxv, *sems):
    esem = sems[:EBUF]
    gsem = sems[EBUF:EBUF + KBUF]
    scsem = sems[EBUF + KBUF:EBUF + 2 * KBUF]
    zsem = sems[EBUF + 2 * KBUF]
    c_ax = lax.axis_index("c")
    s = lax.axis_index("s")
    wid = s * NC + c_ax

    def edata_start(cc, eb):
        pltpu.async_copy(col_hbm.at[wid, cc], ecol.at[eb], esem[eb])
        pltpu.async_copy(row_hbm.at[wid, cc], erow.at[eb], esem[eb])
        pltpu.async_copy(val_hbm.at[wid, cc], evalv.at[eb], esem[eb])

    def edata_wait(eb):
        pltpu.make_async_copy(col_hbm.at[0, 0], ecol.at[eb], esem[eb]).wait()
        pltpu.make_async_copy(row_hbm.at[0, 0], erow.at[eb], esem[eb]).wait()
        pltpu.make_async_copy(val_hbm.at[0, 0], evalv.at[eb], esem[eb]).wait()

    def gather_start(b, eb):
        pltpu.async_copy(x_hbm.at[ecol.at[eb, 0]], rows.at[b], gsem[b])

    def gather_wait(b):
        pltpu.make_async_copy(x_hbm.at[ecol.at[0, 0]], rows.at[b], gsem[b]).wait()

    def scatter_start(b, eb):
        pltpu.async_copy(rows.at[b], acc.at[erow.at[eb, 0]], scsem[b], add=True)

    def scatter_wait(b):
        pltpu.make_async_copy(rows.at[b], acc.at[erow.at[0, 0]], scsem[b]).wait()

    def scale(b, eb):
        rb = rows.at[b]

        def grp(gg, carry2):
            vals16 = evalv[eb, 0, pl.ds(16 * gg, 16)]
            for le in range(16):
                vv = jnp.full((16,), vals16[le], jnp.float32)
                e = 16 * gg + le
                for j in range(H // 16):
                    sl = pl.ds(16 * j, 16)
                    rb[e, sl] = rb[e, sl] * vv
            return carry2
        lax.fori_loop(0, CH // 16, grp, 0)

    for eb in range(KBUF):
        edata_start(eb, eb)

    zbuf = rows.at[0]

    def zrow(i, carry):
        for j in range(H // 16):
            zbuf[i, pl.ds(16 * j, 16)] = jnp.zeros((16,), jnp.float32)
        return carry
    lax.fori_loop(0, ZR, zrow, 0)

    def zchunk(k, carry):
        cidx = s + NS * k
        @pl.when(cidx < NRC)
        def _():
            pltpu.async_copy(zbuf, acc.at[pl.ds(ZR * cidx, ZR)], zsem)
        return carry
    lax.fori_loop(0, (NRC + NS - 1) // NS, zchunk, 0)

    def zdrain(k, carry):
        cidx = s + NS * k
        @pl.when(cidx < NRC)
        def _():
            pltpu.make_async_copy(zbuf, acc.at[pl.ds(0, ZR)], zsem).wait()
        return carry
    lax.fori_loop(0, (NRC + NS - 1) // NS, zdrain, 0)
    plsc.subcore_barrier()

    edata_wait(0)
    gather_start(0, 0)
    edata_wait(1)
    gather_start(1, 1)

    def grpblk(g, carry):
        for bb in range(EBUF):
            b = bb % KBUF
            if bb < 2:
                @pl.when(g >= 1)
                def _():
                    scatter_wait((b + 2) % KBUF)
            else:
                scatter_wait((b + 2) % KBUF)
            edata_wait((bb + 2) % EBUF)
            gather_start((b + 2) % KBUF, (bb + 2) % EBUF)
            edata_start(EBUF * g + bb + 4, (bb + 4) % EBUF)
            gather_wait(b)
            scale(b, bb)
            scatter_start(b, bb)
        return carry
    lax.fori_loop(0, NB8, grpblk, 0)

    for cc in range(EBUF * NB8, NCHUNK):
        b = cc % KBUF
        eb = cc % EBUF
        scatter_wait((b + 2) % KBUF)
        if cc + 2 < NCHUNK:
            edata_wait((eb + 2) % EBUF)
            gather_start((b + 2) % KBUF, (eb + 2) % EBUF)
        if cc + 4 < NCHUNK:
            edata_start(cc + 4, (eb + 4) % EBUF)
        gather_wait(b)
        scale(b, eb)
        scatter_start(b, eb)
    scatter_wait(3)
    scatter_wait(0)
    plsc.subcore_barrier()

    if write_full:
        def wchunk(k, carry):
            cidx = s + NS * k
            @pl.when(cidx < NRC)
            def _():
                pltpu.async_copy(acc.at[pl.ds(ZR * cidx, ZR)],
                                 out_hbm.at[c_ax, pl.ds(ZR * cidx, ZR)], zsem)
            return carry
        lax.fori_loop(0, (NRC + NS - 1) // NS, wchunk, 0)

        def wdrain(k, carry):
            cidx = s + NS * k
            @pl.when(cidx < NRC)
            def _():
                pltpu.make_async_copy(acc.at[pl.ds(0, ZR)],
                                      out_hbm.at[c_ax, pl.ds(0, ZR)],
                                      zsem).wait()
            return carry
        lax.fori_loop(0, (NRC + NS - 1) // NS, wdrain, 0)
    else:
        gview = rows.at[0, pl.ds(0, GPT)]
        pltpu.sync_copy(idx_hbm.at[pl.ds(s * GPT, GPT)], idxv)
        pltpu.sync_copy(acc.at[idxv], gview)
        pltpu.sync_copy(gview, out_hbm.at[c_ax, pl.ds(s * GPT, GPT)])


def _make_spmm(write_full):
    out_rows = N if write_full else 1024
    return functools.partial(
        pl.kernel,
        mesh=_mesh,
        out_type=jax.ShapeDtypeStruct((NC, out_rows, H), jnp.float32),
        scratch_types=[
            pltpu.VMEM_SHARED((N, H), jnp.float32),
            pltpu.VMEM((EBUF, 1, CH), jnp.int32),
            pltpu.VMEM((EBUF, 1, CH), jnp.int32),
            pltpu.VMEM((EBUF, 1, CH), jnp.float32),
            pltpu.VMEM((KBUF, CH, H), jnp.float32),
            pltpu.VMEM((GPT,), jnp.int32),
        ] + [pltpu.SemaphoreType.DMA] * (EBUF + 2 * KBUF + 1),
    )(functools.partial(_spmm_body, write_full))


_spmm_full = _make_spmm(True)
_spmm_gather = _make_spmm(False)


def _tc1_body(f_ref, w_ref, b_ref, o_ref):
    o_ref[...] = jnp.dot(f_ref[...], w_ref[...],
                         preferred_element_type=jnp.float32) + b_ref[...]


def _tc2_body(p_ref, s_ref, h_ref, w_ref, o_ref):
    x = p_ref[0] + p_ref[1]
    y = jnp.maximum(x * s_ref[...] + h_ref[...], 0.0)
    o_ref[...] = jnp.dot(y, w_ref[...], preferred_element_type=jnp.float32)


def _tc3_body(g_ref, s_ref, h_ref, w_ref, b_ref, o_ref):
    x = g_ref[0] + g_ref[1]
    y = jnp.maximum(x * s_ref[...] + h_ref[...], 0.0)
    o_ref[...] = jnp.dot(y, w_ref[...],
                         preferred_element_type=jnp.float32) + b_ref[...]


_RB = 2000


def kernel(features, edge_index, edge_vals, idx, W0, b0, bl0, gamma0, beta0,
           mean0, var0, W1, bl1, gamma1, beta1, mean1, var1, Wf, bf):
    row = edge_index[0].reshape(NW, NCHUNK, 1, CH)
    col = edge_index[1].reshape(NW, NCHUNK, 1, CH)
    val = edge_vals.reshape(NW, NCHUNK, 1, CH)

    scale0 = (gamma0 * lax.rsqrt(var0 + EPS)).reshape(1, H)
    shift0 = ((bl0 - mean0) * gamma0 * lax.rsqrt(var0 + EPS) + beta0).reshape(1, H)
    scale1 = (gamma1 * lax.rsqrt(var1 + EPS)).reshape(1, H)
    shift1 = ((bl1 - mean1) * gamma1 * lax.rsqrt(var1 + EPS) + beta1).reshape(1, H)

    x1 = pl.pallas_call(
        _tc1_body,
        grid=(N // _RB,),
        in_specs=[
            pl.BlockSpec((_RB, D), lambda i: (i, 0)),
            pl.BlockSpec((D, H), lambda i: (0, 0)),
            pl.BlockSpec((1, H), lambda i: (0, 0)),
        ],
        out_specs=pl.BlockSpec((_RB, H), lambda i: (i, 0)),
        out_shape=jax.ShapeDtypeStruct((N, H), jnp.float32),
    )(features, W0, b0.reshape(1, H))

    p1 = _spmm_full(x1, row, col, val, idx)

    x2 = pl.pallas_call(
        _tc2_body,
        grid=(N // _RB,),
        in_specs=[
            pl.BlockSpec((NC, _RB, H), lambda i: (0, i, 0)),
            pl.BlockSpec((1, H), lambda i: (0, 0)),
            pl.BlockSpec((1, H), lambda i: (0, 0)),
            pl.BlockSpec((H, H), lambda i: (0, 0)),
        ],
        out_specs=pl.BlockSpec((_RB, H), lambda i: (i, 0)),
        out_shape=jax.ShapeDtypeStruct((N, H), jnp.float32),
    )(p1, scale0, shift0, W1)

    g = _spmm_gather(x2, row, col, val, idx)

    out = pl.pallas_call(
        _tc3_body,
        in_specs=[
            pl.BlockSpec((NC, 1024, H), lambda: (0, 0, 0)),
            pl.BlockSpec((1, H), lambda: (0, 0)),
            pl.BlockSpec((1, H), lambda: (0, 0)),
            pl.BlockSpec((H, NLAB), lambda: (0, 0)),
            pl.BlockSpec((1, NLAB), lambda: (0, 0)),
        ],
        out_specs=pl.BlockSpec((1024, NLAB), lambda: (0, 0)),
        out_shape=jax.ShapeDtypeStruct((1024, NLAB), jnp.float32),
    )(g, scale1, shift1, Wf, bf.reshape(1, NLAB))

    return out
